# Initial kernel scaffold; baseline (speedup 1.0000x reference)
import jax
import jax.numpy as jnp
from jax.experimental import pallas as pl


def kernel(q, k, v, simthreshd1, cdfthreshd):
    raise NotImplementedError("write your pallas kernel here")


# speedup vs baseline: 1.6450x; 1.6450x over previous
import functools

import jax
import jax.numpy as jnp
from jax.experimental import pallas as pl
from jax.experimental.pallas import tpu as pltpu

BLK = 64
NEG = -1e9


def _mask_kernel(s1_ref, cdf_ref, q_ref, k_ref, bias_ref):
    h = pl.program_id(0)
    q = q_ref[0]
    k = k_ref[0]
    S, D = q.shape
    nb = S // BLK
    scale = 1.0 / (D ** 0.5)

    qb = q.reshape(nb, BLK, D)
    qmean = jnp.mean(qb, axis=1)
    qn = qb / (jnp.sqrt(jnp.sum(qb * qb, axis=-1, keepdims=True)) + 1e-6)
    mn = qmean / (jnp.sqrt(jnp.sum(qmean * qmean, axis=-1, keepdims=True)) + 1e-6)
    cos = jnp.sum(qn * mn[:, None, :], axis=-1)
    block_sim = jnp.min(cos, axis=-1)

    kg = jnp.mean(k, axis=0, keepdims=True)
    kmean = jnp.mean(k.reshape(nb, BLK, D), axis=1) - kg
    est = jax.lax.dot_general(qmean, kmean, (((1,), (1,)), ((), ())),
                              preferred_element_type=jnp.float32) * scale
    m = jnp.max(est, axis=-1, keepdims=True)
    e = jnp.exp(est - m)
    p = e / jnp.sum(e, axis=-1, keepdims=True)

    p_l = p[:, :, None]
    p_j = p[:, None, :]
    lidx = jax.lax.broadcasted_iota(jnp.int32, (nb, nb, nb), 1)
    jidx = jax.lax.broadcasted_iota(jnp.int32, (nb, nb, nb), 2)
    before = (p_l > p_j) | ((p_l == p_j) & (lidx < jidx))
    cumbefore = jnp.sum(jnp.where(before, p_l, 0.0), axis=1)

    keep = cumbefore < (1.0 - cdf_ref[h])
    keep = keep | (block_sim <= s1_ref[h])[:, None]

    bias_small = jnp.where(keep, 0.0, NEG)
    bid = jax.lax.broadcasted_iota(jnp.int32, (nb, S), 0)
    jid = jax.lax.broadcasted_iota(jnp.int32, (nb, S), 1) // BLK
    rk = (bid == jid).astype(jnp.float32)
    bias_ref[0] = jax.lax.dot_general(
        bias_small, rk, (((1,), (0,)), ((), ())),
        preferred_element_type=jnp.float32)


def _attn_kernel(q_ref, k_ref, v_ref, bias_ref, o_ref):
    q = q_ref[0]
    k = k_ref[0]
    v = v_ref[0]
    bias = bias_ref[0]
    QB, D = q.shape
    S = k.shape[0]
    qbb = QB // BLK
    scale = 1.0 / (D ** 0.5)

    s = jax.lax.dot_general(q, k, (((1,), (1,)), ((), ())),
                            preferred_element_type=jnp.float32) * scale
    rid = jax.lax.broadcasted_iota(jnp.int32, (QB, qbb), 0) // BLK
    cid = jax.lax.broadcasted_iota(jnp.int32, (QB, qbb), 1)
    rq = (rid == cid).astype(jnp.float32)
    s = s + jax.lax.dot_general(rq, bias, (((1,), (0,)), ((), ())),
                                preferred_element_type=jnp.float32)
    m = jnp.max(s, axis=-1, keepdims=True)
    e = jnp.exp(s - m)
    p = e / jnp.sum(e, axis=-1, keepdims=True)
    o_ref[0] = jax.lax.dot_general(p, v, (((1,), (0,)), ((), ())),
                                   preferred_element_type=jnp.float32)


@functools.partial(jax.jit, static_argnames=())
def kernel(q, k, v, simthreshd1, cdfthreshd):
    B, H, S, D = q.shape
    nb = S // BLK
    QB = 512
    nq = S // QB

    qh = q[0]
    kh = k[0]
    vh = v[0]

    bias = pl.pallas_call(
        _mask_kernel,
        grid=(H,),
        in_specs=[
            pl.BlockSpec(memory_space=pltpu.SMEM),
            pl.BlockSpec(memory_space=pltpu.SMEM),
            pl.BlockSpec((1, S, D), lambda h: (h, 0, 0)),
            pl.BlockSpec((1, S, D), lambda h: (h, 0, 0)),
        ],
        out_specs=pl.BlockSpec((1, nb, S), lambda h: (h, 0, 0)),
        out_shape=jax.ShapeDtypeStruct((H, nb, S), jnp.float32),
    )(simthreshd1, cdfthreshd, qh, kh)

    out = pl.pallas_call(
        _attn_kernel,
        grid=(H, nq),
        in_specs=[
            pl.BlockSpec((1, QB, D), lambda h, i: (h, i, 0)),
            pl.BlockSpec((1, S, D), lambda h, i: (h, 0, 0)),
            pl.BlockSpec((1, S, D), lambda h, i: (h, 0, 0)),
            pl.BlockSpec((1, QB // BLK, S), lambda h, i: (h, i, 0)),
        ],
        out_specs=pl.BlockSpec((1, QB, D), lambda h, i: (h, i, 0)),
        out_shape=jax.ShapeDtypeStruct((H, S, D), jnp.float32),
    )(qh, kh, vh, bias)

    return out[None]


# speedup vs baseline: 1.6851x; 1.0243x over previous
import functools

import jax
import jax.numpy as jnp
from jax.experimental import pallas as pl
from jax.experimental.pallas import tpu as pltpu

BLK = 64
NEG = -1e9


def _mask_kernel(s1_ref, cdf_ref, q_ref, k_ref, bias_ref):
    h = pl.program_id(0)
    q = q_ref[0, 0]
    k = k_ref[0, 0]
    S, D = q.shape
    nb = S // BLK
    scale = 1.0 / (D ** 0.5)

    qb = q.reshape(nb, BLK, D)
    qmean = jnp.mean(qb, axis=1)
    qn = qb / (jnp.sqrt(jnp.sum(qb * qb, axis=-1, keepdims=True)) + 1e-6)
    mn = qmean / (jnp.sqrt(jnp.sum(qmean * qmean, axis=-1, keepdims=True)) + 1e-6)
    cos = jnp.sum(qn * mn[:, None, :], axis=-1)
    block_sim = jnp.min(cos, axis=-1)

    kg = jnp.mean(k, axis=0, keepdims=True)
    kmean = jnp.mean(k.reshape(nb, BLK, D), axis=1) - kg
    est = jax.lax.dot_general(qmean, kmean, (((1,), (1,)), ((), ())),
                              preferred_element_type=jnp.float32) * scale
    m = jnp.max(est, axis=-1, keepdims=True)
    e = jnp.exp(est - m)
    p = e / jnp.sum(e, axis=-1, keepdims=True)

    p_l = p[:, :, None]
    p_j = p[:, None, :]
    lidx = jax.lax.broadcasted_iota(jnp.int32, (nb, nb, nb), 1)
    jidx = jax.lax.broadcasted_iota(jnp.int32, (nb, nb, nb), 2)
    before = (p_l > p_j) | ((p_l == p_j) & (lidx < jidx))
    cumbefore = jnp.sum(jnp.where(before, p_l, 0.0), axis=1)

    keep = cumbefore < (1.0 - cdf_ref[h])
    keep = keep | (block_sim <= s1_ref[h])[:, None]

    bias_small = jnp.where(keep, 0.0, NEG)
    bid = jax.lax.broadcasted_iota(jnp.int32, (nb, S), 0)
    jid = jax.lax.broadcasted_iota(jnp.int32, (nb, S), 1) // BLK
    rk = (bid == jid).astype(jnp.float32)
    bias_ref[0] = jax.lax.dot_general(
        bias_small, rk, (((1,), (0,)), ((), ())),
        preferred_element_type=jnp.float32)


def _attn_kernel(q_ref, k_ref, v_ref, bias_ref, o_ref):
    q = q_ref[0, 0]
    k = k_ref[0, 0]
    v = v_ref[0, 0]
    bias = bias_ref[0]
    QB, D = q.shape
    S = k.shape[0]
    qbb = QB // BLK
    scale = 1.0 / (D ** 0.5)

    s = jax.lax.dot_general(q, k, (((1,), (1,)), ((), ())),
                            preferred_element_type=jnp.float32) * scale
    rid = jax.lax.broadcasted_iota(jnp.int32, (QB, qbb), 0) // BLK
    cid = jax.lax.broadcasted_iota(jnp.int32, (QB, qbb), 1)
    rq = (rid == cid).astype(jnp.float32)
    s = s + jax.lax.dot_general(rq, bias, (((1,), (0,)), ((), ())),
                                preferred_element_type=jnp.float32)
    m = jnp.max(s, axis=-1, keepdims=True)
    e = jnp.exp(s - m)
    p = e / jnp.sum(e, axis=-1, keepdims=True)
    o_ref[0, 0] = jax.lax.dot_general(p, v, (((1,), (0,)), ((), ())),
                                      preferred_element_type=jnp.float32)


@functools.partial(jax.jit, static_argnames=())
def kernel(q, k, v, simthreshd1, cdfthreshd):
    B, H, S, D = q.shape
    nb = S // BLK
    QB = 512
    nq = S // QB

    bias = pl.pallas_call(
        _mask_kernel,
        grid=(H,),
        in_specs=[
            pl.BlockSpec(memory_space=pltpu.SMEM),
            pl.BlockSpec(memory_space=pltpu.SMEM),
            pl.BlockSpec((1, 1, S, D), lambda h: (0, h, 0, 0)),
            pl.BlockSpec((1, 1, S, D), lambda h: (0, h, 0, 0)),
        ],
        out_specs=pl.BlockSpec((1, nb, S), lambda h: (h, 0, 0)),
        out_shape=jax.ShapeDtypeStruct((H, nb, S), jnp.float32),
    )(simthreshd1, cdfthreshd, q, k)

    out = pl.pallas_call(
        _attn_kernel,
        grid=(H, nq),
        in_specs=[
            pl.BlockSpec((1, 1, QB, D), lambda h, i: (0, h, i, 0)),
            pl.BlockSpec((1, 1, S, D), lambda h, i: (0, h, 0, 0)),
            pl.BlockSpec((1, 1, S, D), lambda h, i: (0, h, 0, 0)),
            pl.BlockSpec((1, QB // BLK, S), lambda h, i: (h, i, 0)),
        ],
        out_specs=pl.BlockSpec((1, 1, QB, D), lambda h, i: (0, h, i, 0)),
        out_shape=jax.ShapeDtypeStruct((B, H, S, D), jnp.float32),
    )(q, k, v, bias)

    return out


# speedup vs baseline: 1.8996x; 1.1273x over previous
import functools

import jax
import jax.numpy as jnp
from jax.experimental import pallas as pl
from jax.experimental.pallas import tpu as pltpu

BLK = 64
NEG = -1e9


def _mask_kernel(s1_ref, cdf_ref, q_ref, k_ref, bias_ref):
    h = pl.program_id(0)
    q = q_ref[0, 0]
    k = k_ref[0, 0]
    S, D = q.shape
    nb = S // BLK
    scale = 1.0 / (D ** 0.5)

    qb = q.reshape(nb, BLK, D)
    qmean = jnp.mean(qb, axis=1)
    qn = qb / (jnp.sqrt(jnp.sum(qb * qb, axis=-1, keepdims=True)) + 1e-6)
    mn = qmean / (jnp.sqrt(jnp.sum(qmean * qmean, axis=-1, keepdims=True)) + 1e-6)
    cos = jnp.sum(qn * mn[:, None, :], axis=-1)
    block_sim = jnp.min(cos, axis=-1)

    kg = jnp.mean(k, axis=0, keepdims=True)
    kmean = jnp.mean(k.reshape(nb, BLK, D), axis=1) - kg
    est = jax.lax.dot_general(qmean, kmean, (((1,), (1,)), ((), ())),
                              preferred_element_type=jnp.float32) * scale
    m = jnp.max(est, axis=-1, keepdims=True)
    e = jnp.exp(est - m)
    p = e / jnp.sum(e, axis=-1, keepdims=True)

    p_l = p[:, :, None]
    p_j = p[:, None, :]
    lidx = jax.lax.broadcasted_iota(jnp.int32, (nb, nb, nb), 1)
    jidx = jax.lax.broadcasted_iota(jnp.int32, (nb, nb, nb), 2)
    before = (p_l > p_j) | ((p_l == p_j) & (lidx < jidx))
    cumbefore = jnp.sum(jnp.where(before, p_l, 0.0), axis=1)

    keep = cumbefore < (1.0 - cdf_ref[h])
    keep = keep | (block_sim <= s1_ref[h])[:, None]

    bias_small = jnp.where(keep, 0.0, NEG)
    bid = jax.lax.broadcasted_iota(jnp.int32, (nb, S), 0)
    jid = jax.lax.broadcasted_iota(jnp.int32, (nb, S), 1) // BLK
    rk = (bid == jid).astype(jnp.float32)
    bias_ref[0] = jax.lax.dot_general(
        bias_small, rk, (((1,), (0,)), ((), ())),
        preferred_element_type=jnp.float32)


def _attn_kernel(q_ref, k_ref, v_ref, bias_ref, o_ref):
    q = q_ref[0, 0]
    k = k_ref[0, 0]
    v = v_ref[0, 0]
    bias = bias_ref[0]
    QB, D = q.shape
    S = k.shape[0]
    qbb = QB // BLK
    scale = 1.0 / (D ** 0.5)

    s = jax.lax.dot_general(q, k, (((1,), (1,)), ((), ())),
                            preferred_element_type=jnp.float32) * scale
    s = (s.reshape(qbb, BLK, S) + bias[:, None, :]).reshape(QB, S)
    m = jnp.max(s, axis=-1, keepdims=True)
    e = jnp.exp(s - m)
    p = e / jnp.sum(e, axis=-1, keepdims=True)
    o_ref[0, 0] = jax.lax.dot_general(p, v, (((1,), (0,)), ((), ())),
                                      preferred_element_type=jnp.float32)


@functools.partial(jax.jit, static_argnames=())
def kernel(q, k, v, simthreshd1, cdfthreshd):
    B, H, S, D = q.shape
    nb = S // BLK
    QB = 512
    nq = S // QB

    bias = pl.pallas_call(
        _mask_kernel,
        grid=(H,),
        in_specs=[
            pl.BlockSpec(memory_space=pltpu.SMEM),
            pl.BlockSpec(memory_space=pltpu.SMEM),
            pl.BlockSpec((1, 1, S, D), lambda h: (0, h, 0, 0)),
            pl.BlockSpec((1, 1, S, D), lambda h: (0, h, 0, 0)),
        ],
        out_specs=pl.BlockSpec((1, nb, S), lambda h: (h, 0, 0)),
        out_shape=jax.ShapeDtypeStruct((H, nb, S), jnp.float32),
    )(simthreshd1, cdfthreshd, q, k)

    out = pl.pallas_call(
        _attn_kernel,
        grid=(H, nq),
        in_specs=[
            pl.BlockSpec((1, 1, QB, D), lambda h, i: (0, h, i, 0)),
            pl.BlockSpec((1, 1, S, D), lambda h, i: (0, h, 0, 0)),
            pl.BlockSpec((1, 1, S, D), lambda h, i: (0, h, 0, 0)),
            pl.BlockSpec((1, QB // BLK, S), lambda h, i: (h, i, 0)),
        ],
        out_specs=pl.BlockSpec((1, 1, QB, D), lambda h, i: (0, h, i, 0)),
        out_shape=jax.ShapeDtypeStruct((B, H, S, D), jnp.float32),
    )(q, k, v, bias)

    return out
